# baseline (speedup 1.0000x reference)
import functools

import jax
import jax.numpy as jnp
from jax.experimental import pallas as pl


def _logits_tile(x_ref, w_ref, b_ref, out_ref):
    x = x_ref[...]
    w = w_ref[...].astype(jnp.bfloat16)
    acc = jax.lax.dot_general(
        x, w,
        dimension_numbers=(((1,), (1,)), ((), ())),
        preferred_element_type=jnp.float32,
    )
    out_ref[...] = acc + b_ref[...]


@functools.partial(jax.jit, static_argnames=("block_n",))
def _lsh_logits(inputs, W, b, block_n=2048):
    batch, d = inputs.shape
    n = W.shape[0]
    x16 = inputs.astype(jnp.bfloat16)
    b2d = b.reshape(1, n)
    grid = (pl.cdiv(n, block_n),)
    return pl.pallas_call(
        _logits_tile,
        grid=grid,
        in_specs=[
            pl.BlockSpec((batch, d), lambda j: (0, 0)),
            pl.BlockSpec((block_n, d), lambda j: (j, 0)),
            pl.BlockSpec((1, block_n), lambda j: (0, j)),
        ],
        out_specs=pl.BlockSpec((batch, block_n), lambda j: (0, j)),
        out_shape=jax.ShapeDtypeStruct((batch, n), jnp.float32),
    )(x16, W, b2d)


def kernel(inputs, labels, freeze, slide, W, b):
    return _lsh_logits(inputs, W, b, block_n=512)


# speedup vs baseline: 1.1770x; 1.1770x over previous
import functools

import jax
import jax.numpy as jnp
from jax.experimental import pallas as pl


def _logits_tile(x_ref, w_ref, b_ref, out_ref):
    x = x_ref[...]
    w = w_ref[...].astype(jnp.bfloat16)
    acc = jax.lax.dot_general(
        x, w,
        dimension_numbers=(((1,), (1,)), ((), ())),
        preferred_element_type=jnp.float32,
    )
    out_ref[...] = acc + b_ref[...]


@functools.partial(jax.jit, static_argnames=("block_n",))
def _lsh_logits(inputs, W, b, block_n=2048):
    batch, d = inputs.shape
    n = W.shape[0]
    x16 = inputs.astype(jnp.bfloat16)
    b2d = b.reshape(1, n)
    grid = (pl.cdiv(n, block_n),)
    return pl.pallas_call(
        _logits_tile,
        grid=grid,
        in_specs=[
            pl.BlockSpec((batch, d), lambda j: (0, 0)),
            pl.BlockSpec((block_n, d), lambda j: (j, 0)),
            pl.BlockSpec((1, block_n), lambda j: (0, j)),
        ],
        out_specs=pl.BlockSpec((batch, block_n), lambda j: (0, j)),
        out_shape=jax.ShapeDtypeStruct((batch, n), jnp.float32),
    )(x16, W, b2d)


def kernel(inputs, labels, freeze, slide, W, b):
    return _lsh_logits(inputs, W, b, block_n=4096)
